# baseline (speedup 1.0000x reference)
import functools

import jax
import jax.numpy as jnp
from jax import lax
from jax.experimental import pallas as pl
from jax.experimental.pallas import tpu as pltpu
from jax.experimental.pallas import tpu_sc as plsc

VOCAB = 1000
BATCH = 4096
HIST = 20
PRIOR = 1.0


def _table_body(n_ref, pt_ref):
    n = n_ref[...] + PRIOR
    s = jnp.sum(n, axis=1, keepdims=True)
    p = jnp.log(n / s)
    p = jnp.maximum(p, -1.0e6)
    pt_ref[...] = p.T


def _compute_table_t(N):
    return pl.pallas_call(
        _table_body,
        out_shape=jax.ShapeDtypeStruct((VOCAB, VOCAB), jnp.float32),
        in_specs=[pl.BlockSpec(memory_space=pltpu.VMEM)],
        out_specs=pl.BlockSpec(memory_space=pltpu.VMEM),
    )(N)


def _make_tgather():
    info = plsc.get_sparse_core_info()
    NC, NS = info.num_cores, info.num_subcores
    NW = NC * NS
    L = info.num_lanes
    NVT = VOCAB // 8
    NU = -(-NVT // NW)
    NIT = NU * HIST
    NBG = BATCH // L
    mesh = plsc.VectorSubcoreMesh(core_axis_name="c", subcore_axis_name="s")

    @functools.partial(
        pl.kernel,
        mesh=mesh,
        out_type=jax.ShapeDtypeStruct((HIST, VOCAB, BATCH), jnp.float32),
        scratch_types=[
            pltpu.VMEM((8, VOCAB), jnp.float32),
            pltpu.VMEM((BATCH,), jnp.int32),
            pltpu.VMEM((8, BATCH), jnp.float32),
            pltpu.VMEM((8, BATCH), jnp.float32),
            pltpu.SemaphoreType.DMA,
            pltpu.SemaphoreType.DMA,
        ],
        compiler_params=pltpu.CompilerParams(needs_layout_passes=False),
    )
    def tgather(pt_hbm, idxt_hbm, out_hbm, slab, idxrow, ob0, ob1, so0, so1):
        wid = lax.axis_index("s") * NC + lax.axis_index("c")
        vis = [jnp.full((L,), vi, jnp.int32) for vi in range(8)]

        def wait_out(sem, ob):
            pltpu.make_async_copy(ob, out_hbm.at[0, pl.ds(0, 8)],
                                  sem).wait()

        def unit(it, ob, sem):
            u = it // HIST
            h = it - u * HIST
            vt = u * NW + wid

            @pl.when(vt < NVT)
            def _():
                @pl.when(h == 0)
                def _():
                    pltpu.sync_copy(pt_hbm.at[pl.ds(vt * 8, 8)], slab)
                pltpu.sync_copy(idxt_hbm.at[pl.ds(h * BATCH, BATCH)],
                                idxrow)

                @pl.when(it >= 2)
                def _():
                    wait_out(sem, ob)

                def bg_body(bg, carry):
                    iv = idxrow[pl.ds(bg * L, L)]
                    for vi in range(8):
                        vals = plsc.load_gather(slab, [vis[vi], iv])
                        ob[vi, pl.ds(bg * L, L)] = vals
                    return carry

                lax.fori_loop(0, NBG, bg_body, 0)
                pltpu.async_copy(ob, out_hbm.at[h, pl.ds(vt * 8, 8)], sem)

        def body(k, carry):
            unit(2 * k, ob0, so0)
            unit(2 * k + 1, ob1, so1)
            return carry

        lax.fori_loop(0, NIT // 2, body, 0)
        wait_out(so0, ob0)
        wait_out(so1, ob1)

    return tgather


_tgather = _make_tgather()


def kernel(N, idx):
    pt = _compute_table_t(N.astype(jnp.float32))
    idxt = jnp.transpose(idx.astype(jnp.int32)).reshape(-1)
    t = _tgather(pt, idxt)
    return jnp.transpose(t, (2, 0, 1))


# speedup vs baseline: 3.3492x; 3.3492x over previous
import functools

import jax
import jax.numpy as jnp
from jax import lax
from jax.experimental import pallas as pl
from jax.experimental.pallas import tpu as pltpu
from jax.experimental.pallas import tpu_sc as plsc

VOCAB = 1000
BATCH = 4096
HIST = 20
PRIOR = 1.0


def _table_body(n_ref, pt_ref):
    n = n_ref[...] + PRIOR
    s = jnp.sum(n, axis=1, keepdims=True)
    p = jnp.log(n / s)
    p = jnp.maximum(p, -1.0e6)
    pt_ref[...] = p.T


def _compute_table_t(N):
    return pl.pallas_call(
        _table_body,
        out_shape=jax.ShapeDtypeStruct((VOCAB, VOCAB), jnp.float32),
        in_specs=[pl.BlockSpec(memory_space=pltpu.VMEM)],
        out_specs=pl.BlockSpec(memory_space=pltpu.VMEM),
    )(N)


def _make_tgather():
    info = plsc.get_sparse_core_info()
    NC, NS = info.num_cores, info.num_subcores
    NW = NC * NS
    L = info.num_lanes
    NVT = VOCAB // 8
    NU = -(-NVT // NW)
    NIT = NU * HIST
    NBG = BATCH // L
    mesh = plsc.VectorSubcoreMesh(core_axis_name="c", subcore_axis_name="s")

    @functools.partial(
        pl.kernel,
        mesh=mesh,
        out_type=jax.ShapeDtypeStruct((HIST, VOCAB, BATCH), jnp.float32),
        scratch_types=[
            pltpu.VMEM((8, VOCAB), jnp.float32),
            pltpu.VMEM((BATCH,), jnp.int32),
            pltpu.VMEM((8, BATCH), jnp.float32),
            pltpu.VMEM((8, BATCH), jnp.float32),
            pltpu.SemaphoreType.DMA,
            pltpu.SemaphoreType.DMA,
        ],
        compiler_params=pltpu.CompilerParams(needs_layout_passes=False),
    )
    def tgather(pt_hbm, idxt_hbm, out_hbm, slab, idxrow, ob0, ob1, so0, so1):
        wid = lax.axis_index("s") * NC + lax.axis_index("c")
        vis = [jnp.full((L,), vi, jnp.int32) for vi in range(8)]

        def wait_out(sem, ob):
            pltpu.make_async_copy(ob, out_hbm.at[0, pl.ds(0, 8)],
                                  sem).wait()

        def unit(it, ob, sem):
            u = it // HIST
            h = it - u * HIST
            vt = u * NW + wid

            @pl.when(vt < NVT)
            def _():
                @pl.when(h == 0)
                def _():
                    pltpu.sync_copy(pt_hbm.at[pl.ds(vt * 8, 8)], slab)
                pltpu.sync_copy(idxt_hbm.at[pl.ds(h * BATCH, BATCH)],
                                idxrow)

                @pl.when(it >= 2)
                def _():
                    wait_out(sem, ob)

                @plsc.parallel_loop(0, BATCH, L * 2, unroll=8)
                def _(b):
                    for sub in range(2):
                        iv = idxrow[pl.ds(b + sub * L, L)]
                        for vi in range(8):
                            vals = plsc.load_gather(slab, [vis[vi], iv])
                            ob[vi, pl.ds(b + sub * L, L)] = vals
                pltpu.async_copy(ob, out_hbm.at[h, pl.ds(vt * 8, 8)], sem)

        def body(k, carry):
            unit(2 * k, ob0, so0)
            unit(2 * k + 1, ob1, so1)
            return carry

        lax.fori_loop(0, NIT // 2, body, 0)
        wait_out(so0, ob0)
        wait_out(so1, ob1)

    return tgather


_tgather = _make_tgather()


def kernel(N, idx):
    pt = _compute_table_t(N.astype(jnp.float32))
    idxt = jnp.transpose(idx.astype(jnp.int32)).reshape(-1)
    t = _tgather(pt, idxt)
    return jnp.transpose(t, (2, 0, 1))
